# baseline (speedup 1.0000x reference)
import functools

import jax
import jax.numpy as jnp
from jax import lax
from jax.experimental import pallas as pl
from jax.experimental.pallas import tpu as pltpu
from jax.experimental.pallas import tpu_sc as plsc

N_MASKS = 100000
N_PROTOTYPES = 64
BATCH = 16384

_NC = 2
_NS = 16
_NW = _NC * _NS
_LANES = 16

_BPW = BATCH // _NW
_CHUNK = 128
_NCHUNK = _BPW // _CHUNK
_CCH = N_PROTOTYPES // _LANES

_mesh = plsc.VectorSubcoreMesh(core_axis_name="c", subcore_axis_name="s")


@functools.partial(
    pl.kernel,
    out_type=jax.ShapeDtypeStruct((BATCH, N_PROTOTYPES), jnp.float32),
    mesh=_mesh,
    scratch_types=[
        pltpu.VMEM((_NCHUNK, _CHUNK), jnp.int32),
        pltpu.VMEM((_BPW, N_PROTOTYPES), jnp.float32),
        pltpu.VMEM((_BPW, N_PROTOTYPES), jnp.float32),
        pltpu.SemaphoreType.DMA,
    ],
    compiler_params=pltpu.CompilerParams(use_tc_tiling_on_sc=False),
)
def _proto_mul(table_hbm, idx_hbm, in_hbm, out_hbm, idx_v, rows_v, in_v, sem):
    wid = lax.axis_index("s") * _NC + lax.axis_index("c")
    base = wid * _BPW

    pltpu.sync_copy(idx_hbm.at[pl.ds(wid * _NCHUNK, _NCHUNK)], idx_v)
    copies = [
        pltpu.async_copy(
            table_hbm.at[idx_v.at[j]],
            rows_v.at[pl.ds(j * _CHUNK, _CHUNK)],
            sem,
        )
        for j in range(_NCHUNK)
    ]
    pltpu.sync_copy(in_hbm.at[pl.ds(base, _BPW)], in_v)
    for cp in copies:
        cp.wait()

    @plsc.parallel_loop(0, _BPW)
    def _row(r):
        for c in range(_CCH):
            sl = pl.ds(c * _LANES, _LANES)
            x = rows_v[r, sl]
            y = in_v[r, sl]
            rows_v[r, sl] = y / (1.0 + jnp.exp(-x))

    pltpu.sync_copy(rows_v, out_hbm.at[pl.ds(base, _BPW)])


def kernel(in_repr, mask_idx, prototype_knobs):
    idx2d = mask_idx.astype(jnp.int32).reshape(_NW * _NCHUNK, _CHUNK)
    return _proto_mul(prototype_knobs, idx2d, in_repr)


# speedup vs baseline: 1.9839x; 1.9839x over previous
import functools

import jax
import jax.numpy as jnp
from jax import lax
from jax.experimental import pallas as pl
from jax.experimental.pallas import tpu as pltpu
from jax.experimental.pallas import tpu_sc as plsc

N_MASKS = 100000
N_PROTOTYPES = 64
BATCH = 16384

_NC = 2
_NS = 16
_NW = _NC * _NS
_LANES = 16

_FPW = N_PROTOTYPES // _NW
_HALF = BATCH // 2
_NB = _HALF // _LANES

_mesh = plsc.VectorSubcoreMesh(core_axis_name="c", subcore_axis_name="s")


@functools.partial(
    pl.kernel,
    out_type=jax.ShapeDtypeStruct((N_PROTOTYPES, BATCH), jnp.float32),
    mesh=_mesh,
    scratch_types=[
        pltpu.VMEM((N_MASKS,), jnp.float32),
        pltpu.VMEM((_HALF,), jnp.int32),
        pltpu.VMEM((_HALF,), jnp.float32),
        pltpu.SemaphoreType.DMA,
    ],
    compiler_params=pltpu.CompilerParams(
        use_tc_tiling_on_sc=True, needs_layout_passes=False
    ),
)
def _proto_mul(table_hbm, idx_hbm, in_hbm, out_hbm, plane_v, idx_v, io_v, sem):
    wid = lax.axis_index("s") * _NC + lax.axis_index("c")

    for fp in range(_FPW):
        f = wid + fp * _NW
        pltpu.sync_copy(table_hbm.at[f], plane_v)
        for h in range(2):
            pltpu.sync_copy(idx_hbm.at[pl.ds(h * _HALF, _HALF)], idx_v)
            pltpu.sync_copy(in_hbm.at[f, pl.ds(h * _HALF, _HALF)], io_v)

            @plsc.parallel_loop(0, _NB)
            def _blk(bb):
                sl = pl.ds(bb * _LANES, _LANES)
                iv = idx_v[sl]
                x = plsc.load_gather(plane_v, [iv])
                y = io_v[sl]
                io_v[sl] = y / (1.0 + jnp.exp(-x))

            pltpu.sync_copy(io_v, out_hbm.at[f, pl.ds(h * _HALF, _HALF)])


def kernel(in_repr, mask_idx, prototype_knobs):
    out_t = _proto_mul(prototype_knobs.T, mask_idx.astype(jnp.int32), in_repr.T)
    return out_t.T


# speedup vs baseline: 2.6876x; 1.3547x over previous
import functools

import jax
import jax.numpy as jnp
from jax import lax
from jax.experimental import pallas as pl
from jax.experimental.pallas import tpu as pltpu
from jax.experimental.pallas import tpu_sc as plsc

N_MASKS = 100000
N_PROTOTYPES = 64
BATCH = 16384

_NC = 2
_NS = 16
_NW = _NC * _NS
_LANES = 16

_FPW = N_PROTOTYPES // _NW
_NQ = 4
_QB = BATCH // _NQ
_NB = _QB // _LANES

_mesh = plsc.VectorSubcoreMesh(core_axis_name="c", subcore_axis_name="s")


@functools.partial(
    pl.kernel,
    out_type=jax.ShapeDtypeStruct((N_PROTOTYPES, BATCH), jnp.float32),
    mesh=_mesh,
    scratch_types=[
        pltpu.VMEM((N_MASKS,), jnp.float32),
        pltpu.VMEM((BATCH,), jnp.int32),
        pltpu.VMEM((_QB,), jnp.float32),
        pltpu.VMEM((_QB,), jnp.float32),
        pltpu.SemaphoreType.DMA,
        pltpu.SemaphoreType.DMA,
        pltpu.SemaphoreType.DMA,
        pltpu.SemaphoreType.DMA,
        pltpu.SemaphoreType.DMA,
    ],
    compiler_params=pltpu.CompilerParams(
        use_tc_tiling_on_sc=True, needs_layout_passes=False
    ),
)
def _proto_mul(
    table_hbm, idx_hbm, in_hbm, out_hbm,
    plane_v, idx_v, io0, io1, sem_p, sem_i, sem_l, sem_s0, sem_s1,
):
    wid = lax.axis_index("s") * _NC + lax.axis_index("c")
    bufs = (io0, io1)
    ssems = (sem_s0, sem_s1)

    idx_cp = pltpu.async_copy(idx_hbm, idx_v, sem_i)

    for fp in range(_FPW):
        f = wid + fp * _NW
        plane_cp = pltpu.async_copy(table_hbm.at[f], plane_v, sem_p)
        loads = [pltpu.async_copy(in_hbm.at[f, pl.ds(0, _QB)], bufs[0], sem_l)]
        stores = [None, None]
        plane_cp.wait()
        if fp == 0:
            idx_cp.wait()

        for q in range(_NQ):
            cur = bufs[q % 2]
            if q + 1 < _NQ:
                nxt = bufs[(q + 1) % 2]
                if stores[(q + 1) % 2] is not None:
                    stores[(q + 1) % 2].wait()
                    stores[(q + 1) % 2] = None
                loads.append(
                    pltpu.async_copy(
                        in_hbm.at[f, pl.ds((q + 1) * _QB, _QB)], nxt, sem_l
                    )
                )
            loads[q].wait()

            qbase = q * _QB

            @plsc.parallel_loop(0, _NB, unroll=4)
            def _blk(bb):
                sl = pl.ds(bb * _LANES, _LANES)
                iv = idx_v[pl.ds(qbase + bb * _LANES, _LANES)]
                x = plsc.load_gather(plane_v, [iv])
                y = cur[sl]
                cur[sl] = y / (1.0 + jnp.exp(-x))

            stores[q % 2] = pltpu.async_copy(
                cur, out_hbm.at[f, pl.ds(qbase, _QB)], ssems[q % 2]
            )
        for s in stores:
            if s is not None:
                s.wait()


def kernel(in_repr, mask_idx, prototype_knobs):
    out_t = _proto_mul(prototype_knobs.T, mask_idx.astype(jnp.int32), in_repr.T)
    return out_t.T


# speedup vs baseline: 2.6914x; 1.0014x over previous
import functools

import jax
import jax.numpy as jnp
from jax import lax
from jax.experimental import pallas as pl
from jax.experimental.pallas import tpu as pltpu
from jax.experimental.pallas import tpu_sc as plsc

N_MASKS = 100000
N_PROTOTYPES = 64
BATCH = 16384

_NC = 2
_NS = 16
_NW = _NC * _NS
_LANES = 16

_FPW = N_PROTOTYPES // _NW
_NQ = 4
_QB = BATCH // _NQ
_NB = _QB // _LANES

_mesh = plsc.VectorSubcoreMesh(core_axis_name="c", subcore_axis_name="s")


@functools.partial(
    pl.kernel,
    out_type=jax.ShapeDtypeStruct((N_PROTOTYPES, BATCH), jnp.float32),
    mesh=_mesh,
    scratch_types=[
        pltpu.VMEM((N_MASKS,), jnp.float32),
        pltpu.VMEM((BATCH,), jnp.int32),
        pltpu.VMEM((_QB,), jnp.float32),
        pltpu.VMEM((_QB,), jnp.float32),
        pltpu.SemaphoreType.DMA,
        pltpu.SemaphoreType.DMA,
        pltpu.SemaphoreType.DMA,
        pltpu.SemaphoreType.DMA,
        pltpu.SemaphoreType.DMA,
    ],
    compiler_params=pltpu.CompilerParams(
        use_tc_tiling_on_sc=True,
        needs_layout_passes=False,
        disable_bounds_checks=True,
        disable_semaphore_checks=True,
        skip_device_barrier=True,
    ),
)
def _proto_mul(
    table_hbm, idx_hbm, in_hbm, out_hbm,
    plane_v, idx_v, io0, io1, sem_p, sem_i, sem_l, sem_s0, sem_s1,
):
    wid = lax.axis_index("s") * _NC + lax.axis_index("c")
    bufs = (io0, io1)
    ssems = (sem_s0, sem_s1)

    idx_cp = pltpu.async_copy(idx_hbm, idx_v, sem_i)

    for fp in range(_FPW):
        f = wid + fp * _NW
        plane_cp = pltpu.async_copy(table_hbm.at[f], plane_v, sem_p)
        loads = [pltpu.async_copy(in_hbm.at[f, pl.ds(0, _QB)], bufs[0], sem_l)]
        stores = [None, None]
        plane_cp.wait()
        if fp == 0:
            idx_cp.wait()

        for q in range(_NQ):
            cur = bufs[q % 2]
            if q + 1 < _NQ:
                nxt = bufs[(q + 1) % 2]
                if stores[(q + 1) % 2] is not None:
                    stores[(q + 1) % 2].wait()
                    stores[(q + 1) % 2] = None
                loads.append(
                    pltpu.async_copy(
                        in_hbm.at[f, pl.ds((q + 1) * _QB, _QB)], nxt, sem_l
                    )
                )
            loads[q].wait()

            qbase = q * _QB

            @plsc.parallel_loop(0, _NB, unroll=4)
            def _blk(bb):
                sl = pl.ds(bb * _LANES, _LANES)
                iv = idx_v[pl.ds(qbase + bb * _LANES, _LANES)]
                x = plsc.load_gather(plane_v, [iv])
                y = cur[sl]
                cur[sl] = y / (1.0 + jnp.exp(-x))

            stores[q % 2] = pltpu.async_copy(
                cur, out_hbm.at[f, pl.ds(qbase, _QB)], ssems[q % 2]
            )
        for s in stores:
            if s is not None:
                s.wait()


def kernel(in_repr, mask_idx, prototype_knobs):
    out_t = _proto_mul(prototype_knobs.T, mask_idx.astype(jnp.int32), in_repr.T)
    return out_t.T


# speedup vs baseline: 2.6995x; 1.0030x over previous
import functools

import jax
import jax.numpy as jnp
from jax import lax
from jax.experimental import pallas as pl
from jax.experimental.pallas import tpu as pltpu
from jax.experimental.pallas import tpu_sc as plsc

N_MASKS = 100000
N_PROTOTYPES = 64
BATCH = 16384

_NC = 2
_NS = 16
_NW = _NC * _NS
_LANES = 16

_FPW = N_PROTOTYPES // _NW
_NQ = 4
_QB = BATCH // _NQ
_NB = _QB // _LANES

_mesh = plsc.VectorSubcoreMesh(core_axis_name="c", subcore_axis_name="s")


@functools.partial(
    pl.kernel,
    out_type=jax.ShapeDtypeStruct((N_PROTOTYPES, BATCH), jnp.float32),
    mesh=_mesh,
    scratch_types=[
        pltpu.VMEM((N_MASKS,), jnp.float32),
        pltpu.VMEM((BATCH,), jnp.int32),
        pltpu.VMEM((_QB,), jnp.float32),
        pltpu.VMEM((_QB,), jnp.float32),
        pltpu.SemaphoreType.DMA,
        pltpu.SemaphoreType.DMA,
        pltpu.SemaphoreType.DMA,
        pltpu.SemaphoreType.DMA,
        pltpu.SemaphoreType.DMA,
    ],
    compiler_params=pltpu.CompilerParams(
        use_tc_tiling_on_sc=True, needs_layout_passes=False
    ),
)
def _proto_mul(
    table_hbm, idx_hbm, in_hbm, out_hbm,
    plane_v, idx_v, io0, io1, sem_p, sem_i, sem_l, sem_s0, sem_s1,
):
    wid = lax.axis_index("s") * _NC + lax.axis_index("c")
    bufs = (io0, io1)
    ssems = (sem_s0, sem_s1)

    idx_cp = pltpu.async_copy(idx_hbm, idx_v, sem_i)

    for fp in range(_FPW):
        f = wid + fp * _NW
        plane_cp = pltpu.async_copy(table_hbm.at[f], plane_v, sem_p)
        loads = [pltpu.async_copy(in_hbm.at[f, pl.ds(0, _QB)], bufs[0], sem_l)]
        stores = [None, None]
        plane_cp.wait()
        if fp == 0:
            idx_cp.wait()

        for q in range(_NQ):
            cur = bufs[q % 2]
            if q + 1 < _NQ:
                nxt = bufs[(q + 1) % 2]
                if stores[(q + 1) % 2] is not None:
                    stores[(q + 1) % 2].wait()
                    stores[(q + 1) % 2] = None
                loads.append(
                    pltpu.async_copy(
                        in_hbm.at[f, pl.ds((q + 1) * _QB, _QB)], nxt, sem_l
                    )
                )
            loads[q].wait()

            qbase = q * _QB

            @plsc.parallel_loop(0, _NB, unroll=8)
            def _blk(bb):
                sl = pl.ds(bb * _LANES, _LANES)
                iv = idx_v[pl.ds(qbase + bb * _LANES, _LANES)]
                x = plsc.load_gather(plane_v, [iv])
                y = cur[sl]
                cur[sl] = y / (1.0 + jnp.exp(-x))

            stores[q % 2] = pltpu.async_copy(
                cur, out_hbm.at[f, pl.ds(qbase, _QB)], ssems[q % 2]
            )
        for s in stores:
            if s is not None:
                s.wait()


def kernel(in_repr, mask_idx, prototype_knobs):
    out_t = _proto_mul(prototype_knobs.T, mask_idx.astype(jnp.int32), in_repr.T)
    return out_t.T


# speedup vs baseline: 2.8172x; 1.0436x over previous
import functools

import jax
import jax.numpy as jnp
from jax import lax
from jax.experimental import pallas as pl
from jax.experimental.pallas import tpu as pltpu
from jax.experimental.pallas import tpu_sc as plsc

N_MASKS = 100000
N_PROTOTYPES = 64
BATCH = 16384

_NC = 2
_NS = 16
_NW = _NC * _NS
_LANES = 16

_FPW = N_PROTOTYPES // _NW
_NQ = 4
_QB = BATCH // _NQ
_NB = _QB // _LANES

_mesh = plsc.VectorSubcoreMesh(core_axis_name="c", subcore_axis_name="s")


@functools.partial(
    pl.kernel,
    out_type=jax.ShapeDtypeStruct((N_PROTOTYPES, BATCH), jnp.float32),
    mesh=_mesh,
    scratch_types=[
        pltpu.VMEM((N_MASKS,), jnp.float32),
        pltpu.VMEM((BATCH,), jnp.int32),
        pltpu.VMEM_SHARED((BATCH,), jnp.int32),
        pltpu.VMEM((_QB,), jnp.float32),
        pltpu.VMEM((_QB,), jnp.float32),
        pltpu.SemaphoreType.DMA,
        pltpu.SemaphoreType.DMA,
        pltpu.SemaphoreType.DMA,
        pltpu.SemaphoreType.DMA,
        pltpu.SemaphoreType.DMA,
    ],
    compiler_params=pltpu.CompilerParams(
        use_tc_tiling_on_sc=True, needs_layout_passes=False
    ),
)
def _proto_mul(
    table_hbm, idx_hbm, in_hbm, out_hbm,
    plane_v, idx_v, idx_sh, io0, io1, sem_p, sem_i, sem_l, sem_s0, sem_s1,
):
    sid = lax.axis_index("s")
    wid = sid * _NC + lax.axis_index("c")
    bufs = (io0, io1)
    ssems = (sem_s0, sem_s1)

    seg = BATCH // _NS
    pltpu.async_copy(
        idx_hbm.at[pl.ds(sid * seg, seg)], idx_sh.at[pl.ds(sid * seg, seg)], sem_i
    ).wait()
    plsc.subcore_barrier()
    idx_cp = pltpu.async_copy(idx_sh, idx_v, sem_i)

    for fp in range(_FPW):
        f = wid + fp * _NW
        plane_cp = pltpu.async_copy(table_hbm.at[f], plane_v, sem_p)
        loads = [pltpu.async_copy(in_hbm.at[f, pl.ds(0, _QB)], bufs[0], sem_l)]
        stores = [None, None]
        plane_cp.wait()
        if fp == 0:
            idx_cp.wait()

        for q in range(_NQ):
            cur = bufs[q % 2]
            if q + 1 < _NQ:
                nxt = bufs[(q + 1) % 2]
                if stores[(q + 1) % 2] is not None:
                    stores[(q + 1) % 2].wait()
                    stores[(q + 1) % 2] = None
                loads.append(
                    pltpu.async_copy(
                        in_hbm.at[f, pl.ds((q + 1) * _QB, _QB)], nxt, sem_l
                    )
                )
            loads[q].wait()

            qbase = q * _QB

            @plsc.parallel_loop(0, _NB, unroll=8)
            def _blk(bb):
                sl = pl.ds(bb * _LANES, _LANES)
                iv = idx_v[pl.ds(qbase + bb * _LANES, _LANES)]
                x = plsc.load_gather(plane_v, [iv])
                y = cur[sl]
                cur[sl] = y / (1.0 + jnp.exp(-x))

            stores[q % 2] = pltpu.async_copy(
                cur, out_hbm.at[f, pl.ds(qbase, _QB)], ssems[q % 2]
            )
        for s in stores:
            if s is not None:
                s.wait()


def kernel(in_repr, mask_idx, prototype_knobs):
    out_t = _proto_mul(prototype_knobs.T, mask_idx.astype(jnp.int32), in_repr.T)
    return out_t.T


# speedup vs baseline: 2.8659x; 1.0173x over previous
import functools

import jax
import jax.numpy as jnp
from jax import lax
from jax.experimental import pallas as pl
from jax.experimental.pallas import tpu as pltpu
from jax.experimental.pallas import tpu_sc as plsc

N_MASKS = 100000
N_PROTOTYPES = 64
BATCH = 16384

_NC = 2
_NS = 16
_NW = _NC * _NS
_LANES = 16

_FPW = N_PROTOTYPES // _NW
_NQ = 4
_QB = BATCH // _NQ
_NB = _QB // _LANES

_mesh = plsc.VectorSubcoreMesh(core_axis_name="c", subcore_axis_name="s")


@functools.partial(
    pl.kernel,
    out_type=jax.ShapeDtypeStruct((N_PROTOTYPES, BATCH), jnp.float32),
    mesh=_mesh,
    scratch_types=[
        pltpu.VMEM((N_MASKS,), jnp.float32),
        pltpu.VMEM((BATCH,), jnp.int32),
        pltpu.VMEM_SHARED((BATCH,), jnp.int32),
        pltpu.VMEM((_QB,), jnp.float32),
        pltpu.VMEM((_QB,), jnp.float32),
        pltpu.SemaphoreType.DMA,
        pltpu.SemaphoreType.DMA,
        pltpu.SemaphoreType.DMA,
        pltpu.SemaphoreType.DMA,
        pltpu.SemaphoreType.DMA,
    ],
    compiler_params=pltpu.CompilerParams(
        use_tc_tiling_on_sc=True, needs_layout_passes=False
    ),
)
def _proto_mul(
    table_hbm, idx_hbm, in_hbm, out_hbm,
    plane_v, idx_v, idx_sh, io0, io1, sem_p, sem_i, sem_l, sem_s0, sem_s1,
):
    sid = lax.axis_index("s")
    wid = sid * _NC + lax.axis_index("c")
    bufs = (io0, io1)
    ssems = (sem_s0, sem_s1)

    plane_cp = pltpu.async_copy(table_hbm.at[wid], plane_v, sem_p)
    first_load = pltpu.async_copy(in_hbm.at[wid, pl.ds(0, _QB)], bufs[0], sem_l)

    seg = BATCH // _NS
    pltpu.async_copy(
        idx_hbm.at[pl.ds(sid * seg, seg)], idx_sh.at[pl.ds(sid * seg, seg)], sem_i
    ).wait()
    plsc.subcore_barrier()
    idx_cp = pltpu.async_copy(idx_sh, idx_v, sem_i)

    for fp in range(_FPW):
        f = wid + fp * _NW
        if fp > 0:
            plane_cp = pltpu.async_copy(table_hbm.at[f], plane_v, sem_p)
            first_load = pltpu.async_copy(
                in_hbm.at[f, pl.ds(0, _QB)], bufs[0], sem_l
            )
        loads = [first_load]
        stores = [None, None]
        plane_cp.wait()
        if fp == 0:
            idx_cp.wait()

        for q in range(_NQ):
            cur = bufs[q % 2]
            if q + 1 < _NQ:
                nxt = bufs[(q + 1) % 2]
                if stores[(q + 1) % 2] is not None:
                    stores[(q + 1) % 2].wait()
                    stores[(q + 1) % 2] = None
                loads.append(
                    pltpu.async_copy(
                        in_hbm.at[f, pl.ds((q + 1) * _QB, _QB)], nxt, sem_l
                    )
                )
            loads[q].wait()

            qbase = q * _QB

            @plsc.parallel_loop(0, _NB, unroll=8)
            def _blk(bb):
                sl = pl.ds(bb * _LANES, _LANES)
                iv = idx_v[pl.ds(qbase + bb * _LANES, _LANES)]
                x = plsc.load_gather(plane_v, [iv])
                y = cur[sl]
                cur[sl] = y / (1.0 + jnp.exp(-x))

            stores[q % 2] = pltpu.async_copy(
                cur, out_hbm.at[f, pl.ds(qbase, _QB)], ssems[q % 2]
            )
        for s in stores:
            if s is not None:
                s.wait()


def kernel(in_repr, mask_idx, prototype_knobs):
    out_t = _proto_mul(prototype_knobs.T, mask_idx.astype(jnp.int32), in_repr.T)
    return out_t.T
